# baseline (speedup 1.0000x reference)
import functools

import jax
import jax.numpy as jnp
from jax import lax
from jax.experimental import pallas as pl
from jax.experimental.pallas import tpu as pltpu
from jax.experimental.pallas import tpu_sc as plsc

VOCAB = 1_000_000
DIM = 64
MODEL_DIM = 768
MOD = VOCAB - 1

BATCH = 4
SEQ = 8192
TOK = BATCH * SEQ
NC = 2
NS = 16
NW = NC * NS
BPW = TOK // NW
NCHUNK = 8
CHUNK = BPW // NCHUNK

_sc_mesh = plsc.VectorSubcoreMesh(core_axis_name="c", subcore_axis_name="s")


@functools.partial(
    pl.kernel,
    mesh=_sc_mesh,
    out_type=jax.ShapeDtypeStruct((TOK, DIM), jnp.bfloat16),
    scratch_types=[
        pltpu.VMEM((BPW + 16,), jnp.int32),
        pltpu.VMEM((NCHUNK, CHUNK), jnp.int32),
        pltpu.VMEM((BPW, DIM), jnp.bfloat16),
        pltpu.SemaphoreType.DMA,
    ],
    compiler_params=pltpu.CompilerParams(use_tc_tiling_on_sc=False),
)
def _sc_hash_gather(ids_hbm, table_hbm, out_hbm, ext_v, idx_v, rows_v, sem):
    wid = lax.axis_index("s") * NC + lax.axis_index("c")
    base = wid * BPW
    pltpu.sync_copy(ids_hbm.at[pl.ds(base, BPW + 8)], ext_v.at[pl.ds(0, BPW + 8)])
    lanes = lax.iota(jnp.int32, 16)
    for j in range(BPW // 16):
        cur = ext_v[pl.ds(8 + 16 * j, 16)]
        prev = ext_v[pl.ds(7 + 16 * j, 16)]
        a = jnp.int32(36313) * cur
        b = jnp.int32(27191) * prev
        x = lax.bitwise_xor(a, b)
        r = lax.rem(x, jnp.int32(MOD))
        r = jnp.where(r < 0, r + jnp.int32(MOD), r)
        pos = base + (16 * j) + lanes
        first = lax.bitwise_and(pos, jnp.int32(SEQ - 1)) == 0
        h = jnp.where(first, jnp.int32(MOD), r)
        idx_v[j // (CHUNK // 16), pl.ds((j % (CHUNK // 16)) * 16, 16)] = h
    copies = []
    for c in range(NCHUNK):
        copies.append(
            pltpu.async_copy(
                table_hbm.at[idx_v.at[c]],
                rows_v.at[pl.ds(c * CHUNK, CHUNK)],
                sem,
            )
        )
    for cp in copies:
        cp.wait()
    pltpu.sync_copy(rows_v, out_hbm.at[pl.ds(base, BPW)])


_MM_BLK = 2048


def _mm_body(emb_ref, projt_ref, scale_ref, out_ref):
    acc = lax.dot_general(
        emb_ref[...],
        projt_ref[...],
        (((1,), (0,)), ((), ())),
        preferred_element_type=jnp.float32,
    )
    out_ref[0] = acc * scale_ref[0]


def _project(emb, projt, scale_arr):
    return pl.pallas_call(
        _mm_body,
        grid=(BATCH, SEQ // _MM_BLK),
        in_specs=[
            pl.BlockSpec(
                (_MM_BLK, DIM),
                lambda i, j: (i * (SEQ // _MM_BLK) + j, 0),
            ),
            pl.BlockSpec((DIM, MODEL_DIM), lambda i, j: (0, 0)),
            pl.BlockSpec(memory_space=pltpu.SMEM),
        ],
        out_specs=pl.BlockSpec((1, _MM_BLK, MODEL_DIM), lambda i, j: (i, j, 0)),
        out_shape=jax.ShapeDtypeStruct((BATCH, SEQ, MODEL_DIM), jnp.float32),
    )(emb, projt, scale_arr)


def kernel(ids, embed_weight, proj_weight, scale):
    ids32 = ids.astype(jnp.int32).reshape(-1)
    ids_pad = jnp.concatenate([jnp.zeros((8,), jnp.int32), ids32])
    table_bf = embed_weight.astype(jnp.bfloat16)
    emb = _sc_hash_gather(ids_pad, table_bf)
    projt = proj_weight.T.astype(jnp.bfloat16)
    scale_arr = jnp.reshape(scale, (1,)).astype(jnp.float32)
    return _project(emb, projt, scale_arr)


# speedup vs baseline: 1.2047x; 1.2047x over previous
import functools

import jax
import jax.numpy as jnp
from jax import lax
from jax.experimental import pallas as pl
from jax.experimental.pallas import tpu as pltpu
from jax.experimental.pallas import tpu_sc as plsc

VOCAB = 1_000_000
DIM = 64
PAIR_DIM = 2 * DIM
MODEL_DIM = 768
MOD = VOCAB - 1

BATCH = 4
SEQ = 8192
TOK = BATCH * SEQ
NC = 2
NS = 16
NW = NC * NS
BPW = TOK // NW
NCHUNK = 8
CHUNK = BPW // NCHUNK
QTOK = BPW // 4

_sc_mesh = plsc.VectorSubcoreMesh(core_axis_name="c", subcore_axis_name="s")


@functools.partial(
    pl.kernel,
    mesh=_sc_mesh,
    out_type=jax.ShapeDtypeStruct((TOK, DIM), jnp.float32),
    scratch_types=[
        pltpu.VMEM((BPW + 16,), jnp.int32),
        pltpu.VMEM((NCHUNK, CHUNK), jnp.int32),
        pltpu.VMEM((BPW,), jnp.int32),
        pltpu.VMEM((QTOK, PAIR_DIM), jnp.float32),
        pltpu.VMEM((QTOK, DIM), jnp.float32),
        pltpu.SemaphoreType.DMA,
    ],
    compiler_params=pltpu.CompilerParams(needs_layout_passes=False),
)
def _sc_hash_gather(
    ids_hbm, table_hbm, out_hbm, ext_v, idx_v, pv_v, rows_v, comp_v, sem
):
    wid = lax.axis_index("s") * NC + lax.axis_index("c")
    base = wid * BPW
    pltpu.sync_copy(ids_hbm.at[pl.ds(base, BPW + 8)], ext_v.at[pl.ds(0, BPW + 8)])
    lanes = lax.iota(jnp.int32, 16)
    for j in range(BPW // 16):
        cur = ext_v[pl.ds(8 + 16 * j, 16)]
        prev = ext_v[pl.ds(7 + 16 * j, 16)]
        a = jnp.int32(36313) * cur
        b = jnp.int32(27191) * prev
        x = lax.bitwise_xor(a, b)
        r = lax.rem(x, jnp.int32(MOD))
        r = jnp.where(r < 0, r + jnp.int32(MOD), r)
        pos = base + (16 * j) + lanes
        first = lax.bitwise_and(pos, jnp.int32(SEQ - 1)) == 0
        h = jnp.where(first, jnp.int32(MOD), r)
        idx_v[j // (CHUNK // 16), pl.ds((j % (CHUNK // 16)) * 16, 16)] = (
            lax.shift_right_logical(h, 1)
        )
        pv_v[pl.ds(16 * j, 16)] = lax.bitwise_and(h, jnp.int32(1))
    for q in range(4):
        copies = []
        for c in range(QTOK // CHUNK):
            chunk = q * (QTOK // CHUNK) + c
            copies.append(
                pltpu.async_copy(
                    table_hbm.at[idx_v.at[chunk]],
                    rows_v.at[pl.ds(c * CHUNK, CHUNK)],
                    sem,
                )
            )
        for cp in copies:
            cp.wait()

        def select(g, carry, q=q):
            tok = g * 16 + lanes
            par = pv_v[pl.ds(q * QTOK + g * 16, 16)]
            for d in range(DIM):
                dd = jnp.full((16,), d, jnp.int32)
                v = plsc.load_gather(rows_v, [tok, par * DIM + dd])
                plsc.store_scatter(comp_v, [tok, dd], v)
            return carry

        lax.fori_loop(0, QTOK // 16, select, 0)
        pltpu.sync_copy(comp_v, out_hbm.at[pl.ds(base + q * QTOK, QTOK)])


_MM_BLK = 2048


def _mm_body(emb_ref, projt_ref, scale_ref, out_ref):
    acc = lax.dot_general(
        emb_ref[...],
        projt_ref[...],
        (((1,), (0,)), ((), ())),
        preferred_element_type=jnp.float32,
    )
    out_ref[0] = acc * scale_ref[0]


def _project(emb, projt, scale_arr):
    return pl.pallas_call(
        _mm_body,
        grid=(BATCH, SEQ // _MM_BLK),
        in_specs=[
            pl.BlockSpec(
                (_MM_BLK, DIM),
                lambda i, j: (i * (SEQ // _MM_BLK) + j, 0),
            ),
            pl.BlockSpec((DIM, MODEL_DIM), lambda i, j: (0, 0)),
            pl.BlockSpec(memory_space=pltpu.SMEM),
        ],
        out_specs=pl.BlockSpec((1, _MM_BLK, MODEL_DIM), lambda i, j: (i, j, 0)),
        out_shape=jax.ShapeDtypeStruct((BATCH, SEQ, MODEL_DIM), jnp.float32),
    )(emb, projt, scale_arr)


def kernel(ids, embed_weight, proj_weight, scale):
    ids32 = ids.astype(jnp.int32).reshape(-1)
    ids_pad = jnp.concatenate([jnp.zeros((8,), jnp.int32), ids32])
    table_pairs = embed_weight.reshape(VOCAB // 2, PAIR_DIM)
    emb = _sc_hash_gather(ids_pad, table_pairs)
    projt = proj_weight.T
    scale_arr = jnp.reshape(scale, (1,)).astype(jnp.float32)
    return _project(emb, projt, scale_arr)


# speedup vs baseline: 2.1319x; 1.7697x over previous
import functools

import jax
import jax.numpy as jnp
from jax import lax
from jax.experimental import pallas as pl
from jax.experimental.pallas import tpu as pltpu
from jax.experimental.pallas import tpu_sc as plsc

VOCAB = 1_000_000
DIM = 64
MODEL_DIM = 768
MOD = VOCAB - 1

BATCH = 4
SEQ = 8192
TOK = BATCH * SEQ
NC = 2
NS = 16
NW = NC * NS
BPW = TOK // NW
NCHUNK = 8
CHUNK = BPW // NCHUNK

_sc_mesh = plsc.VectorSubcoreMesh(core_axis_name="c", subcore_axis_name="s")


@functools.partial(
    pl.kernel,
    mesh=_sc_mesh,
    out_type=jax.ShapeDtypeStruct((TOK, DIM), jnp.float32),
    scratch_types=[
        pltpu.VMEM((BPW + 16,), jnp.int32),
        pltpu.VMEM((BPW,), jnp.int32),
        pltpu.VMEM((CHUNK, DIM), jnp.float32),
        pltpu.SemaphoreType.DMA,
    ],
    compiler_params=pltpu.CompilerParams(needs_layout_passes=False),
)
def _sc_hash_gather(ids_hbm, table_hbm, out_hbm, ext_v, h_v, rows_v, sem):
    wid = lax.axis_index("s") * NC + lax.axis_index("c")
    base = wid * BPW
    pltpu.sync_copy(ids_hbm.at[pl.ds(base, BPW + 8)], ext_v.at[pl.ds(0, BPW + 8)])
    lanes = lax.iota(jnp.int32, 16)
    for j in range(BPW // 16):
        cur = ext_v[pl.ds(8 + 16 * j, 16)]
        prev = ext_v[pl.ds(7 + 16 * j, 16)]
        a = jnp.int32(36313) * cur
        b = jnp.int32(27191) * prev
        x = lax.bitwise_xor(a, b)
        r = lax.rem(x, jnp.int32(MOD))
        r = jnp.where(r < 0, r + jnp.int32(MOD), r)
        pos = base + (16 * j) + lanes
        first = lax.bitwise_and(pos, jnp.int32(SEQ - 1)) == 0
        h = jnp.where(first, jnp.int32(MOD), r)
        h_v[pl.ds(16 * j, 16)] = h
    for c in range(NCHUNK):
        def gather_group(g, carry, c=c):
            v = h_v[pl.ds(c * CHUNK + g * 16, 16)]
            for lane in range(16):
                h = jnp.sum(jnp.where(lanes == lane, v, 0))
                pltpu.async_copy(
                    table_hbm.at[pl.ds(h, 1)],
                    rows_v.at[pl.ds(g * 16 + lane, 1)],
                    sem,
                )
            return carry

        lax.fori_loop(0, CHUNK // 16, gather_group, 0)
        pltpu.make_async_copy(
            table_hbm.at[pl.ds(0, CHUNK)], rows_v, sem
        ).wait()
        pltpu.sync_copy(rows_v, out_hbm.at[pl.ds(base + c * CHUNK, CHUNK)])


_MM_BLK = 2048


def _mm_body(emb_ref, projt_ref, scale_ref, out_ref):
    acc = lax.dot_general(
        emb_ref[...],
        projt_ref[...],
        (((1,), (0,)), ((), ())),
        preferred_element_type=jnp.float32,
    )
    out_ref[0] = acc * scale_ref[0]


def _project(emb, projt, scale_arr):
    return pl.pallas_call(
        _mm_body,
        grid=(BATCH, SEQ // _MM_BLK),
        in_specs=[
            pl.BlockSpec(
                (_MM_BLK, DIM),
                lambda i, j: (i * (SEQ // _MM_BLK) + j, 0),
            ),
            pl.BlockSpec((DIM, MODEL_DIM), lambda i, j: (0, 0)),
            pl.BlockSpec(memory_space=pltpu.SMEM),
        ],
        out_specs=pl.BlockSpec((1, _MM_BLK, MODEL_DIM), lambda i, j: (i, j, 0)),
        out_shape=jax.ShapeDtypeStruct((BATCH, SEQ, MODEL_DIM), jnp.float32),
    )(emb, projt, scale_arr)


def kernel(ids, embed_weight, proj_weight, scale):
    ids32 = ids.astype(jnp.int32).reshape(-1)
    ids_pad = jnp.concatenate([jnp.zeros((8,), jnp.int32), ids32])
    emb = _sc_hash_gather(ids_pad, embed_weight)
    projt = proj_weight.T
    scale_arr = jnp.reshape(scale, (1,)).astype(jnp.float32)
    return _project(emb, projt, scale_arr)


# speedup vs baseline: 2.1952x; 1.0297x over previous
import functools

import jax
import jax.numpy as jnp
from jax import lax
from jax.experimental import pallas as pl
from jax.experimental.pallas import tpu as pltpu
from jax.experimental.pallas import tpu_sc as plsc

VOCAB = 1_000_000
DIM = 64
MODEL_DIM = 768
MOD = VOCAB - 1

BATCH = 4
SEQ = 8192
TOK = BATCH * SEQ
NC = 2
NS = 16
NW = NC * NS
BPW = TOK // NW
NCHUNK = 8
CHUNK = BPW // NCHUNK

_sc_mesh = plsc.VectorSubcoreMesh(core_axis_name="c", subcore_axis_name="s")


@functools.partial(
    pl.kernel,
    mesh=_sc_mesh,
    out_type=jax.ShapeDtypeStruct((TOK,), jnp.int32),
    scratch_types=[
        pltpu.VMEM((BPW + 16,), jnp.int32),
        pltpu.VMEM((BPW,), jnp.int32),
        pltpu.SemaphoreType.DMA,
    ],
    compiler_params=pltpu.CompilerParams(needs_layout_passes=False),
)
def _sc_hash(ids_hbm, h_hbm, ext_v, h_v, sem):
    wid = lax.axis_index("s") * NC + lax.axis_index("c")
    base = wid * BPW
    pltpu.sync_copy(ids_hbm.at[pl.ds(base, BPW + 8)], ext_v.at[pl.ds(0, BPW + 8)])
    lanes = lax.iota(jnp.int32, 16)
    for j in range(BPW // 16):
        cur = ext_v[pl.ds(8 + 16 * j, 16)]
        prev = ext_v[pl.ds(7 + 16 * j, 16)]
        a = jnp.int32(36313) * cur
        b = jnp.int32(27191) * prev
        x = lax.bitwise_xor(a, b)
        r = lax.rem(x, jnp.int32(MOD))
        r = jnp.where(r < 0, r + jnp.int32(MOD), r)
        pos = base + (16 * j) + lanes
        first = lax.bitwise_and(pos, jnp.int32(SEQ - 1)) == 0
        h = jnp.where(first, jnp.int32(MOD), r)
        h_v[pl.ds(16 * j, 16)] = h
    pltpu.sync_copy(h_v, h_hbm.at[pl.ds(base, BPW)])


@functools.partial(
    pl.kernel,
    mesh=_sc_mesh,
    out_type=jax.ShapeDtypeStruct((TOK, DIM), jnp.float32),
    scratch_types=[
        pltpu.VMEM((BPW,), jnp.int32),
        pltpu.VMEM((CHUNK, DIM), jnp.float32),
        pltpu.SemaphoreType.DMA,
    ],
    compiler_params=pltpu.CompilerParams(needs_layout_passes=False),
)
def _sc_gather(h_hbm, table_hbm, out_hbm, h_v, rows_v, sem):
    wid = lax.axis_index("s") * NC + lax.axis_index("c")
    base = wid * BPW
    pltpu.sync_copy(h_hbm.at[pl.ds(base, BPW)], h_v)
    lanes = lax.iota(jnp.int32, 16)
    for c in range(NCHUNK):
        def gather_group(g, carry, c=c):
            v = h_v[pl.ds(c * CHUNK + g * 16, 16)]
            for lane in range(16):
                h = jnp.sum(jnp.where(lanes == lane, v, 0))
                pltpu.async_copy(
                    table_hbm.at[pl.ds(h, 1)],
                    rows_v.at[pl.ds(g * 16 + lane, 1)],
                    sem,
                )
            return carry

        lax.fori_loop(0, CHUNK // 16, gather_group, 0)
        pltpu.make_async_copy(
            table_hbm.at[pl.ds(0, CHUNK)], rows_v, sem
        ).wait()
        pltpu.sync_copy(rows_v, out_hbm.at[pl.ds(base + c * CHUNK, CHUNK)])


_MM_BLK = 2048


def _mm_body(emb_ref, projt_ref, scale_ref, out_ref):
    acc = lax.dot_general(
        emb_ref[...],
        projt_ref[...],
        (((1,), (0,)), ((), ())),
        preferred_element_type=jnp.float32,
    )
    out_ref[0] = acc * scale_ref[0]


def _project(emb, projt, scale_arr):
    return pl.pallas_call(
        _mm_body,
        grid=(BATCH, SEQ // _MM_BLK),
        in_specs=[
            pl.BlockSpec(
                (_MM_BLK, DIM),
                lambda i, j: (i * (SEQ // _MM_BLK) + j, 0),
            ),
            pl.BlockSpec((DIM, MODEL_DIM), lambda i, j: (0, 0)),
            pl.BlockSpec(memory_space=pltpu.SMEM),
        ],
        out_specs=pl.BlockSpec((1, _MM_BLK, MODEL_DIM), lambda i, j: (i, j, 0)),
        out_shape=jax.ShapeDtypeStruct((BATCH, SEQ, MODEL_DIM), jnp.float32),
    )(emb, projt, scale_arr)


def kernel(ids, embed_weight, proj_weight, scale):
    ids32 = ids.astype(jnp.int32).reshape(-1)
    ids_pad = jnp.concatenate([jnp.zeros((8,), jnp.int32), ids32])
    h_all = _sc_hash(ids_pad)
    emb = _sc_gather(h_all, embed_weight)
    projt = proj_weight.T
    scale_arr = jnp.reshape(scale, (1,)).astype(jnp.float32)
    return _project(emb, projt, scale_arr)


# speedup vs baseline: 2.2111x; 1.0073x over previous
import functools

import jax
import jax.numpy as jnp
from jax import lax
from jax.experimental import pallas as pl
from jax.experimental.pallas import tpu as pltpu
from jax.experimental.pallas import tpu_sc as plsc

VOCAB = 1_000_000
DIM = 64
MODEL_DIM = 768
MOD = VOCAB - 1

BATCH = 4
SEQ = 8192
TOK = BATCH * SEQ
NC = 2
NS = 16
NW = NC * NS
BPW = TOK // NW
NCHUNK = 8
CHUNK = BPW // NCHUNK

_sc_mesh = plsc.VectorSubcoreMesh(core_axis_name="c", subcore_axis_name="s")


@functools.partial(
    pl.kernel,
    mesh=_sc_mesh,
    out_type=jax.ShapeDtypeStruct((TOK,), jnp.int32),
    scratch_types=[
        pltpu.VMEM((BPW + 16,), jnp.int32),
        pltpu.VMEM((BPW,), jnp.int32),
        pltpu.SemaphoreType.DMA,
    ],
    compiler_params=pltpu.CompilerParams(needs_layout_passes=False),
)
def _sc_hash(ids_hbm, h_hbm, ext_v, h_v, sem):
    wid = lax.axis_index("s") * NC + lax.axis_index("c")
    base = wid * BPW
    pltpu.sync_copy(ids_hbm.at[pl.ds(base, BPW + 8)], ext_v.at[pl.ds(0, BPW + 8)])
    lanes = lax.iota(jnp.int32, 16)
    for j in range(BPW // 16):
        cur = ext_v[pl.ds(8 + 16 * j, 16)]
        prev = ext_v[pl.ds(7 + 16 * j, 16)]
        a = jnp.int32(36313) * cur
        b = jnp.int32(27191) * prev
        x = lax.bitwise_xor(a, b)
        r = lax.rem(x, jnp.int32(MOD))
        r = jnp.where(r < 0, r + jnp.int32(MOD), r)
        pos = base + (16 * j) + lanes
        first = lax.bitwise_and(pos, jnp.int32(SEQ - 1)) == 0
        h = jnp.where(first, jnp.int32(MOD), r)
        h_v[pl.ds(16 * j, 16)] = h
    pltpu.sync_copy(h_v, h_hbm.at[pl.ds(base, BPW)])


def _make_sc_gather(half):
  BPW2 = BPW // 2
  NCH2 = BPW2 // CHUNK

  @functools.partial(
      pl.kernel,
      mesh=_sc_mesh,
      out_type=jax.ShapeDtypeStruct((TOK // 2, DIM), jnp.float32),
      scratch_types=[
          pltpu.VMEM((BPW2,), jnp.int32),
          pltpu.VMEM((CHUNK, DIM), jnp.float32),
          pltpu.SemaphoreType.DMA,
      ],
      compiler_params=pltpu.CompilerParams(needs_layout_passes=False),
  )
  def _sc_gather(h_hbm, table_hbm, out_hbm, h_v, rows_v, sem):
    wid = lax.axis_index("s") * NC + lax.axis_index("c")
    base = half * (TOK // 2) + wid * BPW2
    obase = wid * BPW2
    pltpu.sync_copy(h_hbm.at[pl.ds(base, BPW2)], h_v)
    lanes = lax.iota(jnp.int32, 16)
    for c in range(NCH2):
      def gather_group(g, carry, c=c):
        v = h_v[pl.ds(c * CHUNK + g * 16, 16)]
        for lane in range(16):
          h = jnp.sum(jnp.where(lanes == lane, v, 0))
          pltpu.async_copy(
              table_hbm.at[pl.ds(h, 1)],
              rows_v.at[pl.ds(g * 16 + lane, 1)],
              sem,
          )
        return carry

      lax.fori_loop(0, CHUNK // 16, gather_group, 0)
      pltpu.make_async_copy(
          table_hbm.at[pl.ds(0, CHUNK)], rows_v, sem
      ).wait()
      pltpu.sync_copy(rows_v, out_hbm.at[pl.ds(obase + c * CHUNK, CHUNK)])

  return _sc_gather


_sc_gather0 = _make_sc_gather(0)
_sc_gather1 = _make_sc_gather(1)


_MM_BLK = 2048


def _mm_body(emb_ref, projt_ref, scale_ref, out_ref):
    acc = lax.dot_general(
        emb_ref[...],
        projt_ref[...],
        (((1,), (0,)), ((), ())),
        preferred_element_type=jnp.float32,
    )
    out_ref[0] = acc * scale_ref[0]


def _mm_body1(emb_ref, projt_ref, scale_ref, prev_ref, out_ref):
    del prev_ref
    _mm_body(emb_ref, projt_ref, scale_ref, out_ref)


def _project0(emb0, projt, scale_arr):
    return pl.pallas_call(
        _mm_body,
        grid=(BATCH // 2, SEQ // _MM_BLK),
        in_specs=[
            pl.BlockSpec(
                (_MM_BLK, DIM),
                lambda i, j: (i * (SEQ // _MM_BLK) + j, 0),
            ),
            pl.BlockSpec((DIM, MODEL_DIM), lambda i, j: (0, 0)),
            pl.BlockSpec(memory_space=pltpu.SMEM),
        ],
        out_specs=pl.BlockSpec((1, _MM_BLK, MODEL_DIM), lambda i, j: (i, j, 0)),
        out_shape=jax.ShapeDtypeStruct((BATCH, SEQ, MODEL_DIM), jnp.float32),
    )(emb0, projt, scale_arr)


def _project1(emb1, projt, scale_arr, prev):
    return pl.pallas_call(
        _mm_body1,
        grid=(BATCH // 2, SEQ // _MM_BLK),
        in_specs=[
            pl.BlockSpec(
                (_MM_BLK, DIM),
                lambda i, j: (i * (SEQ // _MM_BLK) + j, 0),
            ),
            pl.BlockSpec((DIM, MODEL_DIM), lambda i, j: (0, 0)),
            pl.BlockSpec(memory_space=pltpu.SMEM),
            pl.BlockSpec(memory_space=pltpu.HBM),
        ],
        out_specs=pl.BlockSpec(
            (1, _MM_BLK, MODEL_DIM),
            lambda i, j: (i + BATCH // 2, j, 0),
        ),
        out_shape=jax.ShapeDtypeStruct((BATCH, SEQ, MODEL_DIM), jnp.float32),
        input_output_aliases={3: 0},
    )(emb1, projt, scale_arr, prev)


def kernel(ids, embed_weight, proj_weight, scale):
    ids32 = ids.astype(jnp.int32).reshape(-1)
    ids_pad = jnp.concatenate([jnp.zeros((8,), jnp.int32), ids32])
    h_all = _sc_hash(ids_pad)
    emb0 = _sc_gather0(h_all, embed_weight)
    emb1 = _sc_gather1(h_all, embed_weight)
    projt = proj_weight.T
    scale_arr = jnp.reshape(scale, (1,)).astype(jnp.float32)
    out = _project0(emb0, projt, scale_arr)
    return _project1(emb1, projt, scale_arr, out)
